# baseline (speedup 1.0000x reference)
import functools

import jax
import jax.numpy as jnp
from jax import lax
from jax.experimental import pallas as pl
from jax.experimental.pallas import tpu as pltpu
from jax.experimental.pallas import tpu_sc as plsc

N_NODES = 10000
N_EDGES = 160000
DIM_IN = 256
DIM_HALF = 128
NUM_CORES = 2
NUM_SUBCORES = 16
CHUNK = 128
CHUNKS_PER_TILE = 80
N_CHUNKS = CHUNKS_PER_TILE * NUM_SUBCORES
E_PAD = N_CHUNKS * CHUNK
DUMMY_ROW = N_NODES
ACC_ROWS = N_NODES + 16
ROWS_PER_TILE = 624
ROWS_REM = N_NODES - ROWS_PER_TILE * NUM_SUBCORES
ZERO_REM = ACC_ROWS - ROWS_PER_TILE * NUM_SUBCORES


def _mm_body(x_ref, w_ref, o_ref):
    o_ref[...] = lax.dot_general(
        x_ref[...], w_ref[...], (((1,), (1,)), ((), ())),
        preferred_element_type=jnp.float32)


def _matmul_split(x, W):
    m_blk = 1000
    grid = (N_NODES // m_blk, NUM_CORES)
    return pl.pallas_call(
        _mm_body,
        grid=grid,
        in_specs=[
            pl.BlockSpec((m_blk, DIM_IN), lambda i, c: (i, 0)),
            pl.BlockSpec((DIM_HALF, DIM_IN), lambda i, c: (c, 0)),
        ],
        out_specs=pl.BlockSpec(
            (m_blk, DIM_HALF),
            lambda i, c: (c * (N_NODES // m_blk) + i, 0)),
        out_shape=jax.ShapeDtypeStruct((NUM_CORES * N_NODES, DIM_HALF),
                                       jnp.float32),
    )(x, W)


def _sc_aggregate(h2, src2, dst2, zeros):
    mesh = plsc.VectorSubcoreMesh(
        core_axis_name="c", subcore_axis_name="s",
        num_cores=NUM_CORES, num_subcores=NUM_SUBCORES)

    @functools.partial(
        pl.kernel,
        out_type=jax.ShapeDtypeStruct((NUM_CORES * N_NODES, DIM_HALF),
                                      jnp.float32),
        mesh=mesh,
        scratch_types=[
            pltpu.VMEM((CHUNKS_PER_TILE * CHUNK,), jnp.int32),
            pltpu.VMEM((CHUNKS_PER_TILE * CHUNK,), jnp.int32),
            pltpu.VMEM((CHUNK,), jnp.int32),
            pltpu.VMEM((CHUNK,), jnp.int32),
            pltpu.VMEM((CHUNK, DIM_HALF), jnp.float32),
            pltpu.VMEM_SHARED((ACC_ROWS, DIM_HALF), jnp.float32),
            pltpu.SemaphoreType.DMA,
        ],
    )
    def agg(h_hbm, src_hbm, dst_hbm, z_hbm, out_hbm,
            sidx_all, didx_all, sidx, didx, rows, acc, sem):
        c = lax.axis_index("c")
        s = lax.axis_index("s")
        row0 = s * ROWS_PER_TILE
        pltpu.sync_copy(z_hbm.at[pl.ds(0, ROWS_PER_TILE)],
                        acc.at[pl.ds(row0, ROWS_PER_TILE)])

        @pl.when(s == NUM_SUBCORES - 1)
        def _():
            pltpu.sync_copy(
                z_hbm.at[pl.ds(0, ZERO_REM)],
                acc.at[pl.ds(ROWS_PER_TILE * NUM_SUBCORES, ZERO_REM)])

        edges0 = s * CHUNKS_PER_TILE * CHUNK
        pltpu.sync_copy(src_hbm.at[pl.ds(edges0, CHUNKS_PER_TILE * CHUNK)],
                        sidx_all)
        pltpu.sync_copy(dst_hbm.at[pl.ds(edges0, CHUNKS_PER_TILE * CHUNK)],
                        didx_all)

        off = c * N_NODES

        @pl.loop(0, CHUNKS_PER_TILE * CHUNK, step=16)
        def _(k):
            sidx_all[pl.ds(k, 16)] = sidx_all[pl.ds(k, 16)] + off

        plsc.subcore_barrier()

        @pl.loop(0, CHUNKS_PER_TILE)
        def _(q):
            base = q * CHUNK

            @pl.loop(0, CHUNK, step=16)
            def _(k):
                sidx[pl.ds(k, 16)] = sidx_all[pl.ds(base + k, 16)]
                didx[pl.ds(k, 16)] = didx_all[pl.ds(base + k, 16)]

            pltpu.async_copy(h_hbm.at[sidx], rows, sem).wait()
            pltpu.sync_copy(rows, acc.at[didx], add=True)

        plsc.subcore_barrier()
        pltpu.sync_copy(acc.at[pl.ds(row0, ROWS_PER_TILE)],
                        out_hbm.at[pl.ds(c * N_NODES + row0, ROWS_PER_TILE)])

        @pl.when(s == NUM_SUBCORES - 1)
        def _():
            tail0 = ROWS_PER_TILE * NUM_SUBCORES
            pltpu.sync_copy(acc.at[pl.ds(tail0, ROWS_REM)],
                            out_hbm.at[pl.ds(c * N_NODES + tail0, ROWS_REM)])

    return agg(h2, src2, dst2, zeros)


def kernel(x, edge_index, W):
    src = edge_index[0].astype(jnp.int32)
    dst = edge_index[1].astype(jnp.int32)
    pad = E_PAD - N_EDGES
    src2 = jnp.concatenate([src, jnp.zeros((pad,), jnp.int32)])
    dst2 = jnp.concatenate([dst, jnp.full((pad,), DUMMY_ROW, jnp.int32)])
    h2 = _matmul_split(x, W)
    zeros = jnp.zeros((ROWS_PER_TILE, DIM_HALF), jnp.float32)
    out2 = _sc_aggregate(h2, src2, dst2, zeros)
    return jnp.concatenate([out2[:N_NODES], out2[N_NODES:]], axis=1)


# speedup vs baseline: 1.6778x; 1.6778x over previous
import functools

import jax
import jax.numpy as jnp
from jax import lax
from jax.experimental import pallas as pl
from jax.experimental.pallas import tpu as pltpu
from jax.experimental.pallas import tpu_sc as plsc

N_NODES = 10000
N_EDGES = 160000
DIM_IN = 256
DIM_HALF = 128
NUM_CORES = 2
NUM_SUBCORES = 16
CHUNK = 128
N_CHUNKS = N_EDGES // CHUNK
FULL_ROUNDS = N_CHUNKS // NUM_SUBCORES
TAIL = N_CHUNKS - FULL_ROUNDS * NUM_SUBCORES
ROWS_PER_TILE = 624
ROWS_REM = N_NODES - ROWS_PER_TILE * NUM_SUBCORES


def _mm_body(x_ref, w_ref, o_ref):
    o_ref[...] = lax.dot_general(
        x_ref[...], w_ref[...], (((1,), (1,)), ((), ())),
        preferred_element_type=jnp.float32)


def _matmul_split(x, W):
    m_blk = 1000
    grid = (N_NODES // m_blk, NUM_CORES)
    return pl.pallas_call(
        _mm_body,
        grid=grid,
        in_specs=[
            pl.BlockSpec((m_blk, DIM_IN), lambda i, c: (i, 0)),
            pl.BlockSpec((DIM_HALF, DIM_IN), lambda i, c: (c, 0)),
        ],
        out_specs=pl.BlockSpec(
            (m_blk, DIM_HALF),
            lambda i, c: (c * (N_NODES // m_blk) + i, 0)),
        out_shape=jax.ShapeDtypeStruct((NUM_CORES * N_NODES, DIM_HALF),
                                       jnp.float32),
    )(x, W)


def _sc_aggregate(h2, sd_arr, zeros):
    mesh = plsc.VectorSubcoreMesh(
        core_axis_name="c", subcore_axis_name="s",
        num_cores=NUM_CORES, num_subcores=NUM_SUBCORES)

    @functools.partial(
        pl.kernel,
        out_type=jax.ShapeDtypeStruct((NUM_CORES * N_NODES, DIM_HALF),
                                      jnp.float32),
        mesh=mesh,
        scratch_types=[
            pltpu.VMEM((1, 2, CHUNK), jnp.int32),
            pltpu.VMEM((CHUNK, DIM_HALF), jnp.float32),
            pltpu.VMEM_SHARED((N_NODES, DIM_HALF), jnp.float32),
            pltpu.SemaphoreType.DMA,
        ],
    )
    def agg(h_hbm, sd_hbm, z_hbm, out_hbm, sd, rows, acc, sem):
        c = lax.axis_index("c")
        s = lax.axis_index("s")
        row0 = s * ROWS_PER_TILE
        pltpu.sync_copy(z_hbm.at[pl.ds(0, ROWS_PER_TILE)],
                        acc.at[pl.ds(row0, ROWS_PER_TILE)])

        @pl.when(s == NUM_SUBCORES - 1)
        def _():
            pltpu.sync_copy(
                z_hbm.at[pl.ds(0, ROWS_REM)],
                acc.at[pl.ds(ROWS_PER_TILE * NUM_SUBCORES, ROWS_REM)])

        plsc.subcore_barrier()

        h_view = h_hbm.at[pl.ds(c * N_NODES, N_NODES)]

        def process(ci):
            pltpu.sync_copy(sd_hbm.at[pl.ds(ci, 1)], sd)
            pltpu.async_copy(h_view.at[sd.at[0, 0]], rows, sem).wait()
            pltpu.sync_copy(rows, acc.at[sd.at[0, 1]], add=True)

        @pl.loop(0, FULL_ROUNDS)
        def _(j):
            process(j * NUM_SUBCORES + s)

        @pl.when(s < TAIL)
        def _():
            process(FULL_ROUNDS * NUM_SUBCORES + s)

        plsc.subcore_barrier()
        pltpu.sync_copy(acc.at[pl.ds(row0, ROWS_PER_TILE)],
                        out_hbm.at[pl.ds(c * N_NODES + row0, ROWS_PER_TILE)])

        @pl.when(s == NUM_SUBCORES - 1)
        def _():
            tail0 = ROWS_PER_TILE * NUM_SUBCORES
            pltpu.sync_copy(acc.at[pl.ds(tail0, ROWS_REM)],
                            out_hbm.at[pl.ds(c * N_NODES + tail0, ROWS_REM)])

    return agg(h2, sd_arr, zeros)


def kernel(x, edge_index, W):
    src = edge_index[0].astype(jnp.int32)
    dst = edge_index[1].astype(jnp.int32)
    sd = jnp.stack([src.reshape(N_CHUNKS, CHUNK),
                    dst.reshape(N_CHUNKS, CHUNK)], axis=1)
    h2 = _matmul_split(x, W)
    zeros = jnp.zeros((ROWS_PER_TILE, DIM_HALF), jnp.float32)
    out2 = _sc_aggregate(h2, sd, zeros)
    return jnp.concatenate([out2[:N_NODES], out2[N_NODES:]], axis=1)


# speedup vs baseline: 1.7397x; 1.0369x over previous
import functools

import jax
import jax.numpy as jnp
from jax import lax
from jax.experimental import pallas as pl
from jax.experimental.pallas import tpu as pltpu
from jax.experimental.pallas import tpu_sc as plsc

N_NODES = 10000
N_EDGES = 160000
DIM_IN = 256
DIM_HALF = 128
NUM_CORES = 2
NUM_SUBCORES = 16
CHUNK = 128
N_CHUNKS = N_EDGES // CHUNK
FULL_ROUNDS = N_CHUNKS // NUM_SUBCORES
TAIL = N_CHUNKS - FULL_ROUNDS * NUM_SUBCORES
ROWS_PER_TILE = 624
ROWS_REM = N_NODES - ROWS_PER_TILE * NUM_SUBCORES


def _mm_body(x_ref, w_ref, o_ref):
    o_ref[...] = lax.dot_general(
        x_ref[...], w_ref[...], (((1,), (1,)), ((), ())),
        preferred_element_type=jnp.float32)


def _matmul_split(x, W):
    m_blk = 1000
    grid = (N_NODES // m_blk, NUM_CORES)
    return pl.pallas_call(
        _mm_body,
        grid=grid,
        in_specs=[
            pl.BlockSpec((m_blk, DIM_IN), lambda i, c: (i, 0)),
            pl.BlockSpec((DIM_HALF, DIM_IN), lambda i, c: (c, 0)),
        ],
        out_specs=pl.BlockSpec(
            (m_blk, DIM_HALF),
            lambda i, c: (c * (N_NODES // m_blk) + i, 0)),
        out_shape=jax.ShapeDtypeStruct((NUM_CORES * N_NODES, DIM_HALF),
                                       jnp.float32),
    )(x, W)


def _sc_aggregate(h2, sd_arr, zeros):
    mesh = plsc.VectorSubcoreMesh(
        core_axis_name="c", subcore_axis_name="s",
        num_cores=NUM_CORES, num_subcores=NUM_SUBCORES)

    @functools.partial(
        pl.kernel,
        out_type=jax.ShapeDtypeStruct((N_NODES, DIM_IN), jnp.float32),
        mesh=mesh,
        scratch_types=[
            pltpu.VMEM((1, 2, CHUNK), jnp.int32),
            pltpu.VMEM((CHUNK, DIM_HALF), jnp.float32),
            pltpu.VMEM_SHARED((N_NODES, DIM_HALF), jnp.float32),
            pltpu.SemaphoreType.DMA,
        ],
    )
    def agg(h_hbm, sd_hbm, z_hbm, out_hbm, sd, rows, acc, sem):
        c = lax.axis_index("c")
        s = lax.axis_index("s")
        row0 = s * ROWS_PER_TILE
        pltpu.sync_copy(z_hbm.at[pl.ds(0, ROWS_PER_TILE)],
                        acc.at[pl.ds(row0, ROWS_PER_TILE)])

        @pl.when(s == NUM_SUBCORES - 1)
        def _():
            pltpu.sync_copy(
                z_hbm.at[pl.ds(0, ROWS_REM)],
                acc.at[pl.ds(ROWS_PER_TILE * NUM_SUBCORES, ROWS_REM)])

        plsc.subcore_barrier()

        h_view = h_hbm.at[pl.ds(c * N_NODES, N_NODES)]

        def process(ci):
            pltpu.sync_copy(sd_hbm.at[pl.ds(ci, 1)], sd)
            pltpu.async_copy(h_view.at[sd.at[0, 0]], rows, sem).wait()
            pltpu.sync_copy(rows, acc.at[sd.at[0, 1]], add=True)

        @pl.loop(0, FULL_ROUNDS)
        def _(j):
            process(j * NUM_SUBCORES + s)

        @pl.when(s < TAIL)
        def _():
            process(FULL_ROUNDS * NUM_SUBCORES + s)

        plsc.subcore_barrier()
        col0 = c * DIM_HALF
        pltpu.sync_copy(
            acc.at[pl.ds(row0, ROWS_PER_TILE)],
            out_hbm.at[pl.ds(row0, ROWS_PER_TILE), pl.ds(col0, DIM_HALF)])

        @pl.when(s == NUM_SUBCORES - 1)
        def _():
            tail0 = ROWS_PER_TILE * NUM_SUBCORES
            pltpu.sync_copy(
                acc.at[pl.ds(tail0, ROWS_REM)],
                out_hbm.at[pl.ds(tail0, ROWS_REM), pl.ds(col0, DIM_HALF)])

    return agg(h2, sd_arr, zeros)


def kernel(x, edge_index, W):
    src = edge_index[0].astype(jnp.int32)
    dst = edge_index[1].astype(jnp.int32)
    sd = jnp.stack([src.reshape(N_CHUNKS, CHUNK),
                    dst.reshape(N_CHUNKS, CHUNK)], axis=1)
    h2 = _matmul_split(x, W)
    zeros = jnp.zeros((ROWS_PER_TILE, DIM_HALF), jnp.float32)
    return _sc_aggregate(h2, sd, zeros)


# speedup vs baseline: 2.5800x; 1.4830x over previous
import functools

import jax
import jax.numpy as jnp
from jax import lax
from jax.experimental import pallas as pl
from jax.experimental.pallas import tpu as pltpu
from jax.experimental.pallas import tpu_sc as plsc

N_NODES = 10000
N_EDGES = 160000
DIM_IN = 256
DIM_HALF = 128
NUM_CORES = 2
NUM_SUBCORES = 16
CHUNK = 128
N_CHUNKS = N_EDGES // CHUNK
FULL_ROUNDS = N_CHUNKS // NUM_SUBCORES
TAIL = N_CHUNKS - FULL_ROUNDS * NUM_SUBCORES
ROWS_PER_TILE = 624
ROWS_REM = N_NODES - ROWS_PER_TILE * NUM_SUBCORES


def _mm_body(x_ref, w_ref, o_ref):
    o_ref[...] = lax.dot_general(
        x_ref[...], w_ref[...], (((1,), (1,)), ((), ())),
        preferred_element_type=jnp.float32)


def _matmul_split(x, W):
    m_blk = 1000
    grid = (N_NODES // m_blk, NUM_CORES)
    return pl.pallas_call(
        _mm_body,
        grid=grid,
        in_specs=[
            pl.BlockSpec((m_blk, DIM_IN), lambda i, c: (i, 0)),
            pl.BlockSpec((DIM_HALF, DIM_IN), lambda i, c: (c, 0)),
        ],
        out_specs=pl.BlockSpec(
            (m_blk, DIM_HALF),
            lambda i, c: (c * (N_NODES // m_blk) + i, 0)),
        out_shape=jax.ShapeDtypeStruct((NUM_CORES * N_NODES, DIM_HALF),
                                       jnp.float32),
    )(x, W)


def _sc_aggregate(h2, sd_arr, zeros):
    mesh = plsc.VectorSubcoreMesh(
        core_axis_name="c", subcore_axis_name="s",
        num_cores=NUM_CORES, num_subcores=NUM_SUBCORES)

    @functools.partial(
        pl.kernel,
        out_type=jax.ShapeDtypeStruct((N_NODES, DIM_IN), jnp.float32),
        mesh=mesh,
        scratch_types=[
            pltpu.VMEM((1, 2, CHUNK), jnp.int32),
            pltpu.VMEM((1, 2, CHUNK), jnp.int32),
            pltpu.VMEM((CHUNK, DIM_HALF), jnp.float32),
            pltpu.VMEM((CHUNK, DIM_HALF), jnp.float32),
            pltpu.VMEM_SHARED((N_NODES, DIM_HALF), jnp.float32),
            pltpu.SemaphoreType.DMA,
        ],
    )
    def agg(h_hbm, sd_hbm, z_hbm, out_hbm, sd0, sd1, rows0, rows1,
            acc, sem):
        c = lax.axis_index("c")
        s = lax.axis_index("s")
        row0 = s * ROWS_PER_TILE
        pltpu.sync_copy(z_hbm.at[pl.ds(0, ROWS_PER_TILE)],
                        acc.at[pl.ds(row0, ROWS_PER_TILE)])

        @pl.when(s == NUM_SUBCORES - 1)
        def _():
            pltpu.sync_copy(
                z_hbm.at[pl.ds(0, ROWS_REM)],
                acc.at[pl.ds(ROWS_PER_TILE * NUM_SUBCORES, ROWS_REM)])

        plsc.subcore_barrier()

        h_view = h_hbm.at[pl.ds(c * N_NODES, N_NODES)]

        sds = (sd0, sd1)
        rowbs = (rows0, rows1)

        def i_load(q, b):
            pltpu.sync_copy(sd_hbm.at[pl.ds(q * NUM_SUBCORES + s, 1)],
                            sds[b])

        def s_sync(b):
            pltpu.sync_copy(rowbs[b], acc.at[sds[b].at[0, 1]], add=True)

        def step(q, u, first, last):
            desc = pltpu.async_copy(h_view.at[sds[u].at[0, 0]], rowbs[u],
                                    sem)
            if not first:
                s_sync(1 - u)
            if not last:
                i_load(q + 1, 1 - u)
            desc.wait()

        def body(t, first, last):
            step(2 * t, 0, first, False)
            step(2 * t + 1, 1, False, last)

        i_load(0, 0)
        body(0, True, False)

        @pl.loop(1, FULL_ROUNDS // 2 - 1)
        def _(t):
            body(t, False, False)

        body(FULL_ROUNDS // 2 - 1, False, True)
        s_sync(1)

        @pl.when(s < TAIL)
        def _():
            i_load(FULL_ROUNDS, 0)
            pltpu.async_copy(h_view.at[sd0.at[0, 0]], rows0, sem).wait()
            s_sync(0)

        plsc.subcore_barrier()
        col0 = c * DIM_HALF
        pltpu.sync_copy(
            acc.at[pl.ds(row0, ROWS_PER_TILE)],
            out_hbm.at[pl.ds(row0, ROWS_PER_TILE), pl.ds(col0, DIM_HALF)])

        @pl.when(s == NUM_SUBCORES - 1)
        def _():
            tail0 = ROWS_PER_TILE * NUM_SUBCORES
            pltpu.sync_copy(
                acc.at[pl.ds(tail0, ROWS_REM)],
                out_hbm.at[pl.ds(tail0, ROWS_REM), pl.ds(col0, DIM_HALF)])

    return agg(h2, sd_arr, zeros)


def kernel(x, edge_index, W):
    src = edge_index[0].astype(jnp.int32)
    dst = edge_index[1].astype(jnp.int32)
    sd = jnp.stack([src.reshape(N_CHUNKS, CHUNK),
                    dst.reshape(N_CHUNKS, CHUNK)], axis=1)
    h2 = _matmul_split(x, W)
    zeros = jnp.zeros((ROWS_PER_TILE, DIM_HALF), jnp.float32)
    return _sc_aggregate(h2, sd, zeros)
